# baseline (speedup 1.0000x reference)
import functools
import math

import jax
import jax.numpy as jnp
from jax import lax
from jax.experimental import pallas as pl
from jax.experimental.pallas import tpu as pltpu
from jax.experimental.pallas import tpu_sc as plsc

D_MODEL = 64
SCALE = math.sqrt(D_MODEL)
LANES = 16

_NC = 2
_NS = 16
_NW = _NC * _NS
_G = 2
_U = 128 * _G
_NRAW = 3
_NTB = 2


def _make_kernel(S0, S1):
    NB = S0 // 128
    n_units = S1 * NB // _G
    assert n_units % _NW == 0
    u_per_w = n_units // _NW
    b_per_w = u_per_w * _U
    upp = NB // _G

    mesh = plsc.VectorSubcoreMesh(core_axis_name="c", subcore_axis_name="s")

    @functools.partial(
        pl.kernel,
        mesh=mesh,
        out_type=jax.ShapeDtypeStruct((S1, 8, NB, 8, 128), jnp.float32),
        compiler_params=pltpu.CompilerParams(
            use_tc_tiling_on_sc=False, needs_layout_passes=False
        ),
        scratch_types=[
            pltpu.VMEM((b_per_w,), jnp.int32),
        ]
        + [pltpu.VMEM((_U, D_MODEL), jnp.float32) for _ in range(_NRAW)]
        + [pltpu.VMEM((8, _G, 8, 128), jnp.float32) for _ in range(_NTB)]
        + [pltpu.SemaphoreType.DMA for _ in range(_NRAW + _NTB)],
    )
    def gather_scale(idx_hbm, table_hbm, out_hbm, idx_v, *scratch):
        raw = scratch[:_NRAW]
        tbuf = scratch[_NRAW : _NRAW + _NTB]
        gsem = scratch[_NRAW + _NTB : 2 * _NRAW + _NTB]
        ssem = scratch[2 * _NRAW + _NTB :]

        wid = lax.axis_index("s") * _NC + lax.axis_index("c")
        ubase = wid * u_per_w
        pltpu.sync_copy(idx_hbm.at[pl.ds(ubase * _U, b_per_w)], idx_v)

        lane = lax.iota(jnp.int32, LANES)

        def gather(k, sr):
            return pltpu.make_async_copy(
                table_hbm.at[idx_v.at[pl.ds(k * _U, _U)]], raw[sr], gsem[sr]
            )

        def store(k, st):
            gu = ubase + k
            s1 = gu // upp
            bo = gu % upp
            return pltpu.make_async_copy(
                tbuf[st],
                out_hbm.at[s1, :, pl.ds(bo * _G, _G)],
                ssem[st],
            )

        def transpose(sr, st):
            rawr = raw[sr]
            tb = tbuf[st]

            def t_body(t, carry):
                bb = t >> 3
                cb = t & 7
                rowv = lane + ((bb << 7) | (cb << 4))
                csl = pl.ds(cb * LANES, LANES)
                for j in range(D_MODEL):
                    a = j >> 3
                    r = j & 7
                    jvec = lane * 0 + j
                    v = plsc.load_gather(rawr, [rowv, jvec])
                    tb[a, bb, r, csl] = v * SCALE
                return carry

            lax.fori_loop(0, 8 * _G, t_body, 0)

        def step(k, sr, st):
            @pl.when(k + 2 < u_per_w)
            def _():
                gather(k + 2, (sr + 2) % _NRAW).start()

            gather(k, sr).wait()

            @pl.when(k >= _NTB)
            def _():
                store(k - _NTB, st).wait()

            transpose(sr, st)
            store(k, st).start()

        gather(0, 0).start()
        gather(1, 1).start()

        PERIOD = 6
        n_main = (u_per_w // PERIOD) * PERIOD

        def outer(i, carry):
            for jj in range(PERIOD):
                step(i * PERIOD + jj, jj % _NRAW, jj % _NTB)
            return carry

        lax.fori_loop(0, u_per_w // PERIOD, outer, 0)
        for k in range(n_main, u_per_w):
            step(k, k % _NRAW, k % _NTB)

        for k in range(u_per_w - _NTB, u_per_w):
            store(k, k % _NTB).wait()

    return gather_scale


def kernel(x, table):
    S0, S1 = x.shape
    xt = jnp.swapaxes(x, 0, 1)
    idx = xt.reshape(S0 * S1).astype(jnp.int32)
    out5d = _make_kernel(S0, S1)(idx, table)
    return out5d.transpose(2, 4, 0, 1, 3).reshape(S0, S1, D_MODEL)


# speedup vs baseline: 1.3630x; 1.3630x over previous
import functools
import math

import jax
import jax.numpy as jnp
from jax import lax
from jax.experimental import pallas as pl
from jax.experimental.pallas import tpu as pltpu
from jax.experimental.pallas import tpu_sc as plsc

D_MODEL = 64
SCALE = math.sqrt(D_MODEL)
LANES = 16

_NC = 2
_NS = 16
_NW = _NC * _NS
_NBUF = 4
_CH = 256
_BLK = 2048


def _make_gather_kernel(B):
    assert B % (_NW * _CH) == 0
    b_per_w = B // _NW
    nchunks = b_per_w // _CH

    mesh = plsc.VectorSubcoreMesh(core_axis_name="c", subcore_axis_name="s")

    @functools.partial(
        pl.kernel,
        mesh=mesh,
        out_type=jax.ShapeDtypeStruct((B, D_MODEL), jnp.float32),
        compiler_params=pltpu.CompilerParams(
            use_tc_tiling_on_sc=False, needs_layout_passes=False
        ),
        scratch_types=[
            pltpu.VMEM((b_per_w,), jnp.int32),
        ]
        + [pltpu.VMEM((_CH, D_MODEL), jnp.float32) for _ in range(_NBUF)]
        + [pltpu.SemaphoreType.DMA for _ in range(2 * _NBUF)],
    )
    def gather_scale(idx_hbm, table_hbm, out_hbm, idx_v, *bufs_and_sems):
        rows = bufs_and_sems[:_NBUF]
        gsem = bufs_and_sems[_NBUF : 2 * _NBUF]
        ssem = bufs_and_sems[2 * _NBUF :]

        wid = lax.axis_index("s") * _NC + lax.axis_index("c")
        base = wid * b_per_w
        pltpu.sync_copy(idx_hbm.at[pl.ds(base, b_per_w)], idx_v)

        def gather(c, b):
            return pltpu.make_async_copy(
                table_hbm.at[idx_v.at[pl.ds(c * _CH, _CH)]], rows[b], gsem[b]
            )

        def store(c, b):
            return pltpu.make_async_copy(
                rows[b], out_hbm.at[pl.ds(base + c * _CH, _CH)], ssem[b]
            )

        def scale(b):
            def row_body(i, carry):
                for j in range(D_MODEL // LANES):
                    sl = pl.ds(j * LANES, LANES)
                    rows[b][i, sl] = rows[b][i, sl] * SCALE
                return carry

            lax.fori_loop(0, _CH, row_body, 0, unroll=8)

        DIST = _NBUF - 2

        for b in range(DIST):
            gather(b, b).start()

        def outer(i, carry):
            c0 = i * _NBUF
            for b in range(_NBUF):
                c = c0 + b
                g = c + DIST
                gb = (b + DIST) % _NBUF

                @pl.when(jnp.logical_and(g < nchunks, g >= _NBUF))
                def _():
                    store(g - _NBUF, gb).wait()

                @pl.when(g < nchunks)
                def _():
                    gather(g, gb).start()

                gather(c, b).wait()
                scale(b)
                store(c, b).start()
            return carry

        lax.fori_loop(0, nchunks // _NBUF, outer, 0)

        for b in range(_NBUF):
            store(nchunks - _NBUF + b, b).wait()

    return gather_scale


def _tc_transpose_body(rows_ref, eye_ref, out_ref):
    out_ref[...] = lax.dot_general(
        eye_ref[...],
        rows_ref[...],
        dimension_numbers=(((1,), (1,)), ((), ())),
        preferred_element_type=jnp.float32,
    )


def _make_tc_transpose(S0, S1):
    H = S0 // _BLK

    return pl.pallas_call(
        _tc_transpose_body,
        grid=(S1, H),
        in_specs=[
            pl.BlockSpec((_BLK, D_MODEL), lambda i, h: (i * H + h, 0)),
            pl.BlockSpec((D_MODEL, D_MODEL), lambda i, h: (0, 0)),
        ],
        out_specs=pl.BlockSpec((D_MODEL, _BLK), lambda i, h: (i, h)),
        out_shape=jax.ShapeDtypeStruct((S1 * D_MODEL, S0), jnp.float32),
        compiler_params=pltpu.CompilerParams(
            dimension_semantics=("parallel", "parallel")
        ),
    )


def kernel(x, table):
    S0, S1 = x.shape
    B = S0 * S1
    xt = jnp.swapaxes(x, 0, 1)
    idx = xt.reshape(B).astype(jnp.int32)
    rows = _make_gather_kernel(B)(idx, table)
    eye = jnp.eye(D_MODEL, dtype=jnp.float32)
    out2d = _make_tc_transpose(S0, S1)(rows, eye)
    return out2d.reshape(S1, D_MODEL, S0).transpose(2, 0, 1)


# speedup vs baseline: 1.4117x; 1.0357x over previous
import functools
import math

import jax
import jax.numpy as jnp
from jax import lax
from jax.experimental import pallas as pl
from jax.experimental.pallas import tpu as pltpu
from jax.experimental.pallas import tpu_sc as plsc

D_MODEL = 64
SCALE = math.sqrt(D_MODEL)
LANES = 16

_NC = 2
_NS = 16
_NW = _NC * _NS
_NBUF = 4
_CH = 256
_TBLK = 8192


def _make_gather_kernel(B, V):
    assert B % (_NW * _CH) == 0
    b_per_w = B // _NW
    nchunks = b_per_w // _CH

    mesh = plsc.VectorSubcoreMesh(core_axis_name="c", subcore_axis_name="s")

    @functools.partial(
        pl.kernel,
        mesh=mesh,
        out_type=jax.ShapeDtypeStruct((B, D_MODEL), jnp.float32),
        compiler_params=pltpu.CompilerParams(
            use_tc_tiling_on_sc=False, needs_layout_passes=False
        ),
        scratch_types=[
            pltpu.VMEM((b_per_w,), jnp.int32),
        ]
        + [pltpu.VMEM((_CH, D_MODEL), jnp.float32) for _ in range(_NBUF)]
        + [pltpu.SemaphoreType.DMA for _ in range(2 * _NBUF)],
    )
    def gather_scale(idx_hbm, table_hbm, out_hbm, idx_v, *bufs_and_sems):
        rows = bufs_and_sems[:_NBUF]
        gsem = bufs_and_sems[_NBUF : 2 * _NBUF]
        ssem = bufs_and_sems[2 * _NBUF :]

        wid = lax.axis_index("s") * _NC + lax.axis_index("c")
        base = wid * b_per_w
        pltpu.sync_copy(idx_hbm.at[pl.ds(base, b_per_w)], idx_v)

        def gather(c, b):
            return pltpu.make_async_copy(
                table_hbm.at[idx_v.at[pl.ds(c * _CH, _CH)]], rows[b], gsem[b]
            )

        def store(c, b):
            return pltpu.make_async_copy(
                rows[b], out_hbm.at[pl.ds(base + c * _CH, _CH)], ssem[b]
            )

        def scale(b):
            def row_body(i, carry):
                for j in range(D_MODEL // LANES):
                    sl = pl.ds(j * LANES, LANES)
                    rows[b][i, sl] = rows[b][i, sl] * SCALE
                return carry

            lax.fori_loop(0, _CH, row_body, 0, unroll=8)

        DIST = _NBUF - 2

        for b in range(DIST):
            gather(b, b).start()

        def outer(i, carry):
            c0 = i * _NBUF
            for b in range(_NBUF):
                c = c0 + b
                g = c + DIST
                gb = (b + DIST) % _NBUF

                @pl.when(jnp.logical_and(g < nchunks, g >= _NBUF))
                def _():
                    store(g - _NBUF, gb).wait()

                @pl.when(g < nchunks)
                def _():
                    gather(g, gb).start()

                gather(c, b).wait()
                scale(b)
                store(c, b).start()
            return carry

        lax.fori_loop(0, nchunks // _NBUF, outer, 0)

        for b in range(_NBUF):
            store(nchunks - _NBUF + b, b).wait()

    return gather_scale


def _table_relayout_body(tt_ref, eye_ref, out_ref):
    out_ref[...] = lax.dot_general(
        tt_ref[...],
        eye_ref[...],
        dimension_numbers=(((0,), (0,)), ((), ())),
        preferred_element_type=jnp.float32,
    )


def _make_table_relayout(V):
    grid = (V + _TBLK - 1) // _TBLK

    return pl.pallas_call(
        _table_relayout_body,
        grid=(grid,),
        in_specs=[
            pl.BlockSpec((D_MODEL, _TBLK), lambda i: (0, i)),
            pl.BlockSpec((D_MODEL, D_MODEL), lambda i: (0, 0)),
        ],
        out_specs=pl.BlockSpec((_TBLK, D_MODEL), lambda i: (i, 0)),
        out_shape=jax.ShapeDtypeStruct((V, D_MODEL), jnp.float32),
        compiler_params=pltpu.CompilerParams(
            dimension_semantics=("parallel",)
        ),
    )


def _out_relayout_body(rows_ref, eye_ref, out_ref):
    out_ref[...] = lax.dot_general(
        eye_ref[...],
        rows_ref[...],
        dimension_numbers=(((1,), (1,)), ((), ())),
        preferred_element_type=jnp.float32,
    )


def _make_out_relayout(S0, S1):
    return pl.pallas_call(
        _out_relayout_body,
        grid=(S1,),
        in_specs=[
            pl.BlockSpec((S0, D_MODEL), lambda i: (i, 0)),
            pl.BlockSpec((D_MODEL, D_MODEL), lambda i: (0, 0)),
        ],
        out_specs=pl.BlockSpec((D_MODEL, S0), lambda i: (i, 0)),
        out_shape=jax.ShapeDtypeStruct((S1 * D_MODEL, S0), jnp.float32),
        compiler_params=pltpu.CompilerParams(
            dimension_semantics=("arbitrary",)
        ),
    )


def kernel(x, table):
    S0, S1 = x.shape
    B = S0 * S1
    V = table.shape[0]
    xt = jnp.swapaxes(x, 0, 1)
    idx = xt.reshape(B).astype(jnp.int32)
    eye = jnp.eye(D_MODEL, dtype=jnp.float32)
    table_rm = _make_table_relayout(V)(jnp.swapaxes(table, 0, 1), eye)
    rows = _make_gather_kernel(B, V)(idx, table_rm)
    out2d = _make_out_relayout(S0, S1)(rows, eye)
    return out2d.reshape(S1, D_MODEL, S0).transpose(2, 0, 1)


# speedup vs baseline: 1.4736x; 1.0438x over previous
import functools
import math

import jax
import jax.numpy as jnp
from jax import lax
from jax.experimental import pallas as pl
from jax.experimental.pallas import tpu as pltpu
from jax.experimental.pallas import tpu_sc as plsc

D_MODEL = 64
SCALE = math.sqrt(D_MODEL)
LANES = 16

_NC = 2
_NS = 16
_NW = _NC * _NS
_NBUF = 4
_CH = 256
_TBLK = 8192


def _make_gather_kernel(B, V):
    assert B % (_NW * _CH) == 0
    b_per_w = B // _NW
    nchunks = b_per_w // _CH

    mesh = plsc.VectorSubcoreMesh(core_axis_name="c", subcore_axis_name="s")

    @functools.partial(
        pl.kernel,
        mesh=mesh,
        out_type=jax.ShapeDtypeStruct((B, D_MODEL), jnp.float32),
        compiler_params=pltpu.CompilerParams(
            use_tc_tiling_on_sc=False, needs_layout_passes=False
        ),
        scratch_types=[
            pltpu.VMEM((b_per_w,), jnp.int32),
        ]
        + [pltpu.VMEM((_CH, D_MODEL), jnp.float32) for _ in range(_NBUF)]
        + [pltpu.SemaphoreType.DMA for _ in range(2 * _NBUF)],
    )
    def gather_scale(idx_hbm, table_hbm, out_hbm, idx_v, *bufs_and_sems):
        rows = bufs_and_sems[:_NBUF]
        gsem = bufs_and_sems[_NBUF : 2 * _NBUF]
        ssem = bufs_and_sems[2 * _NBUF :]

        wid = lax.axis_index("s") * _NC + lax.axis_index("c")
        base = wid * b_per_w
        pltpu.sync_copy(idx_hbm.at[pl.ds(base, b_per_w)], idx_v)

        def gather(c, b):
            return pltpu.make_async_copy(
                table_hbm.at[idx_v.at[pl.ds(c * _CH, _CH)]], rows[b], gsem[b]
            )

        def store(c, b):
            return pltpu.make_async_copy(
                rows[b], out_hbm.at[pl.ds(base + c * _CH, _CH)], ssem[b]
            )

        def scale(b):
            def row_body(i, carry):
                for j in range(D_MODEL // LANES):
                    sl = pl.ds(j * LANES, LANES)
                    rows[b][i, sl] = rows[b][i, sl] * SCALE
                return carry

            lax.fori_loop(0, _CH, row_body, 0, unroll=8)

        DIST = _NBUF - 2

        for b in range(DIST):
            gather(b, b).start()

        def outer(i, carry):
            c0 = i * _NBUF
            for b in range(_NBUF):
                c = c0 + b
                g = c + DIST
                gb = (b + DIST) % _NBUF

                @pl.when(jnp.logical_and(g < nchunks, g >= _NBUF))
                def _():
                    store(g - _NBUF, gb).wait()

                @pl.when(g < nchunks)
                def _():
                    gather(g, gb).start()

                gather(c, b).wait()
                scale(b)
                store(c, b).start()
            return carry

        lax.fori_loop(0, nchunks // _NBUF, outer, 0)

        for b in range(_NBUF):
            store(nchunks - _NBUF + b, b).wait()

    return gather_scale


def _table_relayout_body(tt_ref, eye_ref, out_ref):
    out_ref[...] = lax.dot_general(
        tt_ref[...],
        eye_ref[...],
        dimension_numbers=(((0,), (0,)), ((), ())),
        preferred_element_type=jnp.float32,
    )


def _make_table_relayout(V):
    grid = (V + _TBLK - 1) // _TBLK

    return pl.pallas_call(
        _table_relayout_body,
        grid=(grid,),
        in_specs=[
            pl.BlockSpec((D_MODEL, _TBLK), lambda i: (0, i)),
            pl.BlockSpec((D_MODEL, D_MODEL), lambda i: (0, 0)),
        ],
        out_specs=pl.BlockSpec((_TBLK, D_MODEL), lambda i: (i, 0)),
        out_shape=jax.ShapeDtypeStruct((V, D_MODEL), jnp.float32),
        compiler_params=pltpu.CompilerParams(
            dimension_semantics=("parallel",)
        ),
    )


def _out_relayout_body(rows_ref, eye_ref, out_ref):
    out_ref[...] = lax.dot_general(
        eye_ref[...],
        rows_ref[...],
        dimension_numbers=(((1,), (1,)), ((), ())),
        preferred_element_type=jnp.float32,
    )


def _make_out_relayout(S0, S1):
    return pl.pallas_call(
        _out_relayout_body,
        grid=(S1,),
        in_specs=[
            pl.BlockSpec((S0, D_MODEL), lambda i: (i, 0)),
            pl.BlockSpec((D_MODEL, D_MODEL), lambda i: (0, 0)),
        ],
        out_specs=pl.BlockSpec((D_MODEL, S0), lambda i: (i, 0)),
        out_shape=jax.ShapeDtypeStruct((S1 * D_MODEL, S0), jnp.float32),
        compiler_params=pltpu.CompilerParams(
            dimension_semantics=("arbitrary",)
        ),
    )


def kernel(x, table):
    S0, S1 = x.shape
    B = S0 * S1
    V = table.shape[0]
    xt = jnp.swapaxes(x, 0, 1)
    idx = xt.reshape(B).astype(jnp.int32)
    eye = jnp.eye(D_MODEL, dtype=jnp.float32)
    rows = _make_gather_kernel(B, V)(idx, table)
    out2d = _make_out_relayout(S0, S1)(rows, eye)
    return out2d.reshape(S1, D_MODEL, S0).transpose(2, 0, 1)
